# Initial kernel scaffold; baseline (speedup 1.0000x reference)
import jax
import jax.numpy as jnp
from jax.experimental import pallas as pl


def kernel(pos, atoms, edge_index, batch, params):
    raise NotImplementedError("write your pallas kernel here")


# speedup vs baseline: 2.4284x; 2.4284x over previous
import functools
import math

import jax
import jax.numpy as jnp
import numpy as np
from jax import lax
from jax.experimental import pallas as pl
from jax.experimental.pallas import tpu as pltpu

_R_MAX = 10.0
_NB = 8
_PC = 5
_D = 32
_NG = 32

_HIGH = jax.lax.Precision.HIGHEST


def _dot(a, b):
    return jax.lax.dot_general(a, b, (((1,), (0,)), ((), ())),
                               precision=_HIGH,
                               preferred_element_type=jnp.float32)


def _norm_eps(sq):
    return jnp.sqrt(jnp.clip(sq, 1e-8, None))


def _scalar_ln(s, g, b):
    mu = jnp.mean(s, axis=-1, keepdims=True)
    var = jnp.mean((s - mu) ** 2, axis=-1, keepdims=True)
    return (s - mu) / jnp.sqrt(var + 1e-5) * g + b


def _node_init_body(atoms_ref, emb_ref, lng_ref, lnb_ref, ws_ref, bs_ref,
                    out_ref):
    a = atoms_ref[...]
    onehot = (a == jax.lax.broadcasted_iota(jnp.int32, (a.shape[0], 64), 1)
              ).astype(jnp.float32)
    hs = _dot(onehot, emb_ref[...])
    t = _scalar_ln(hs, lng_ref[...], lnb_ref[...])
    hs = _dot(t, ws_ref[...]) + bs_ref[...]
    out_ref[:, 0:32] = hs
    out_ref[:, 32:128] = jnp.zeros((hs.shape[0], 96), jnp.float32)


def _edge_pre_body(psrc_ref, pdst_ref, lng_ref, lnb_ref, wh_ref, ws_ref,
                   bs_ref, wv_ref, wsv_ref, bsv_ref, out_ref):
    ps = psrc_ref[...]
    pd = pdst_ref[...]
    vec = ps[:, 0:3] - pd[:, 0:3]
    lsq = jnp.sum(vec * vec, axis=-1, keepdims=True)
    length = jnp.sqrt(lsq)
    safe_l = jnp.where(length > 0.0, length, 1.0)
    unit = jnp.where(length > 0.0, vec / safe_l, 0.0)
    k = ((jax.lax.broadcasted_iota(jnp.int32, (1, _NB), 1) + 1
          ).astype(jnp.float32) * (np.pi / _R_MAX))
    bessel = np.sqrt(2.0 / _R_MAX) * jnp.sin(length * k) / safe_l
    x = length / _R_MAX
    p = float(_PC)
    env = (1.0 - (p + 1.0) * (p + 2.0) / 2.0 * x ** 4 * x
           + p * (p + 2.0) * x ** 5 * x
           - p * (p + 1.0) / 2.0 * x ** 6 * x)
    env = env * (length < _R_MAX).astype(jnp.float32)
    es0 = bessel * env
    usq = jnp.sum(unit * unit, axis=-1, keepdims=True)
    vn0 = jnp.sqrt(jnp.clip(usq, 1e-8, None))
    ev0 = unit / vn0
    es0 = _scalar_ln(es0, lng_ref[...], lnb_ref[...])
    wh = wh_ref[0, 0]
    vh = ev0 * wh
    vhn = _norm_eps(jnp.sum(vh * vh, axis=-1, keepdims=True))
    s_out = (_dot(es0, ws_ref[0:8, :]) + vhn * ws_ref[8, :] + bs_ref[...])
    wv = wv_ref[0, 0]
    gate = jax.nn.sigmoid(_dot(s_out, wsv_ref[...]) + bsv_ref[...])
    ev = vh * wv * gate
    out_ref[:, 0:32] = s_out
    out_ref[:, 32:35] = ev
    out_ref[:, 35:64] = jnp.zeros((s_out.shape[0], 29), jnp.float32)


def _edge_layer_body(gs_ref, gd_ref, ee_ref,
                     wh_s_ref, wh_m_ref, wh_d_ref,
                     ws_a_ref, ws_b_ref, ws_c_ref, ws_v_ref, bs_ref,
                     wv_ref, wsv_ref, bsv_ref,
                     wh2_ref, ws2s_ref, ws2v_ref, bs2_ref, wv2_ref,
                     wsv2_ref, bsv2_ref,
                     wh3_ref, ws3s_ref, ws3v_ref, bs3_ref, wv3_ref,
                     wsv3_ref, bsv3_ref,
                     out_ref):
    gs = gs_ref[...]
    gd = gd_ref[...]
    ee = ee_ref[...]
    hs_s, vs = gs[:, 0:32], (gs[:, 32:64], gs[:, 64:96], gs[:, 96:128])
    hs_d, vd = gd[:, 0:32], (gd[:, 32:64], gd[:, 64:96], gd[:, 96:128])
    es = ee[:, 0:32]
    ev = (ee[:, 32:33], ee[:, 33:34], ee[:, 34:35])

    wh_s, wh_m, wh_d = wh_s_ref[...], wh_m_ref[...], wh_d_ref[...]
    vh = [ _dot(vs[c], wh_s) + ev[c] * wh_m + _dot(vd[c], wh_d)
           for c in range(3)]
    vnsq = vh[0] * vh[0] + vh[1] * vh[1] + vh[2] * vh[2]
    vn = _norm_eps(vnsq)
    s1 = (_dot(hs_s, ws_a_ref[...]) + _dot(es, ws_b_ref[...])
          + _dot(hs_d, ws_c_ref[...]) + _dot(vn, ws_v_ref[...])
          + bs_ref[...])
    gate1 = jax.nn.sigmoid(_dot(s1, wsv_ref[...]) + bsv_ref[...])
    wv1 = wv_ref[...]
    v1 = [_dot(vh[c], wv1) * gate1 for c in range(3)]
    s1 = jnp.maximum(s1, 0.0)

    wh2 = wh2_ref[...]
    vh2 = [_dot(v1[c], wh2) for c in range(3)]
    vn2 = _norm_eps(vh2[0] * vh2[0] + vh2[1] * vh2[1] + vh2[2] * vh2[2])
    s2 = (_dot(s1, ws2s_ref[...]) + _dot(vn2, ws2v_ref[...]) + bs2_ref[...])
    gate2 = jax.nn.sigmoid(_dot(s2, wsv2_ref[...]) + bsv2_ref[...])
    wv2 = wv2_ref[...]
    v2 = [_dot(vh2[c], wv2) * gate2 for c in range(3)]
    s2 = jnp.maximum(s2, 0.0)

    wh3 = wh3_ref[...]
    vh3 = [_dot(v2[c], wh3) for c in range(3)]
    vn3 = _norm_eps(vh3[0] * vh3[0] + vh3[1] * vh3[1] + vh3[2] * vh3[2])
    s3 = (_dot(s2, ws3s_ref[...]) + _dot(vn3, ws3v_ref[...]) + bs3_ref[...])
    gate3 = jax.nn.sigmoid(_dot(s3, wsv3_ref[...]) + bsv3_ref[...])
    wv3 = wv3_ref[...]
    v3 = [_dot(vh3[c], wv3) * gate3 for c in range(3)]

    out_ref[:, 0:32] = s3
    out_ref[:, 32:64] = v3[0]
    out_ref[:, 64:96] = v3[1]
    out_ref[:, 96:128] = v3[2]


def _tuple_ln_planar(s, v, g, b):
    vsq = v[0] * v[0] + v[1] * v[1] + v[2] * v[2]
    vn = jnp.sqrt(jnp.mean(jnp.clip(vsq, 1e-8, None), axis=-1,
                           keepdims=True))
    return _scalar_ln(s, g, b), tuple(v[c] / vn for c in range(3))


def _node_layer_body(tab_ref, p0_ref, p1_ref, c0_ref, c1_ref,
                     ln1g_ref, ln1b_ref,
                     f1wh_ref, f1ws_s_ref, f1ws_v_ref, f1bs_ref, f1wv_ref,
                     f1wsv_ref, f1bsv_ref,
                     f2wh_ref, f2ws_s_ref, f2ws_v_ref, f2bs_ref, f2wv_ref,
                     f2wsv_ref, f2bsv_ref,
                     ln2g_ref, ln2b_ref,
                     out_ref):
    tab = tab_ref[...]
    hs = tab[:, 0:32]
    hv = (tab[:, 32:64], tab[:, 64:96], tab[:, 96:128])
    agg = p0_ref[...] + p1_ref[...]
    cnt = jnp.maximum(c0_ref[:, 0:1] + c1_ref[:, 0:1], 1.0)
    ds = agg[:, 0:32] / cnt
    dv = (agg[:, 32:64] / cnt, agg[:, 64:96] / cnt,
          agg[:, 96:128] / cnt)
    hs, hv = _tuple_ln_planar(hs + ds,
                              tuple(hv[c] + dv[c] for c in range(3)),
                              ln1g_ref[...], ln1b_ref[...])
    f1wh = f1wh_ref[...]
    vh = [_dot(hv[c], f1wh) for c in range(3)]
    vn = _norm_eps(vh[0] * vh[0] + vh[1] * vh[1] + vh[2] * vh[2])
    fs = (_dot(hs, f1ws_s_ref[...]) + _dot(vn, f1ws_v_ref[...])
          + f1bs_ref[...])
    gate = jax.nn.sigmoid(_dot(fs, f1wsv_ref[...]) + f1bsv_ref[...])
    f1wv = f1wv_ref[...]
    fv = [_dot(vh[c], f1wv) * gate for c in range(3)]
    fs = jnp.maximum(fs, 0.0)
    f2wh = f2wh_ref[...]
    vh2 = [_dot(fv[c], f2wh) for c in range(3)]
    vn2 = _norm_eps(vh2[0] * vh2[0] + vh2[1] * vh2[1] + vh2[2] * vh2[2])
    fs2 = (_dot(fs, f2ws_s_ref[...]) + _dot(vn2, f2ws_v_ref[...])
           + f2bs_ref[...])
    gate2 = jax.nn.sigmoid(_dot(fs2, f2wsv_ref[...]) + f2bsv_ref[...])
    f2wv = f2wv_ref[...]
    fv2 = [_dot(vh2[c], f2wv) * gate2 for c in range(3)]
    hs, hv = _tuple_ln_planar(hs + fs2,
                              tuple(hv[c] + fv2[c] for c in range(3)),
                              ln2g_ref[...], ln2b_ref[...])
    out_ref[:, 0:32] = hs
    out_ref[:, 32:64] = hv[0]
    out_ref[:, 64:96] = hv[1]
    out_ref[:, 96:128] = hv[2]


def _head_body(tab_ref, bat_ref,
               lng_ref, lnb_ref, wh_ref, ws_s_ref, ws_v_ref, bs_ref,
               w1_ref, b1_ref, w2_ref, b2_ref,
               pool_ref, res_ref):
    i = pl.program_id(0)
    tab = tab_ref[...]
    hs = tab[:, 0:32]
    hv = (tab[:, 32:64], tab[:, 64:96], tab[:, 96:128])
    hs, hv = _tuple_ln_planar(hs, hv, lng_ref[...], lnb_ref[...])
    wh = wh_ref[...]
    vh = [_dot(hv[c], wh) for c in range(3)]
    vn = _norm_eps(vh[0] * vh[0] + vh[1] * vh[1] + vh[2] * vh[2])
    out = (_dot(hs, ws_s_ref[...]) + _dot(vn, ws_v_ref[...]) + bs_ref[...])
    out = jnp.maximum(out, 0.0)
    bat = bat_ref[...]
    onehot = (bat == jax.lax.broadcasted_iota(
        jnp.int32, (bat.shape[0], _NG), 1)).astype(jnp.float32)
    contrib = jax.lax.dot_general(onehot, out, (((0,), (0,)), ((), ())),
                                  precision=_HIGH,
                                  preferred_element_type=jnp.float32)

    @pl.when(i == 0)
    def _():
        pool_ref[...] = jnp.zeros_like(pool_ref)

    pool_ref[...] += contrib

    @pl.when(i == pl.num_programs(0) - 1)
    def _():
        pooled = pool_ref[...]
        h = jnp.maximum(_dot(pooled, w1_ref[...]) + b1_ref[...], 0.0)
        res = _dot(h, w2_ref[...]) + b2_ref[...]
        res_ref[...] = res


def _const_spec(shape):
    return pl.BlockSpec(shape, lambda i: tuple(0 for _ in shape))


def _row_spec(blk, width):
    return pl.BlockSpec((blk, width), lambda i: (i, 0))


def _pc(body, grid, in_specs, out_specs, out_shape, interpret=False):
    return pl.pallas_call(body, grid=(grid,), in_specs=in_specs,
                          out_specs=out_specs, out_shape=out_shape,
                          interpret=interpret)


_INTERPRET = False


def kernel(pos, atoms, edge_index, batch, params):
    n = pos.shape[0]
    e = edge_index.shape[1]
    src, dst = edge_index[0], edge_index[1]

    interp = _INTERPRET
    f32 = jnp.float32

    bn = 1000
    atoms2 = atoms.reshape(n, 1)
    wv_gvp = params['Wv']['gvp']
    tab0 = _pc(
        _node_init_body, n // bn,
        [_row_spec(bn, 1), _const_spec((64, 32)), _const_spec((1, 32)),
         _const_spec((1, 32)), _const_spec((32, 32)), _const_spec((1, 32))],
        _row_spec(bn, 128), jax.ShapeDtypeStruct((n, 128), f32),
        interpret=interp,
    )(atoms2, params['emb_in'],
      params['Wv']['ln_g'].reshape(1, 32), params['Wv']['ln_b'].reshape(1, 32),
      wv_gvp['ws'], wv_gvp['bs'].reshape(1, 32))

    pos16 = jnp.pad(pos, ((0, 0), (0, 13)))
    psrc = pos16[src]
    pdst = pos16[dst]
    we = params['We']
    weg = we['gvp']
    be = 2000
    esev = _pc(
        _edge_pre_body, e // be,
        [_row_spec(be, 16), _row_spec(be, 16),
         _const_spec((1, 8)), _const_spec((1, 8)),
         _const_spec((1, 1)), _const_spec((9, 32)), _const_spec((1, 32)),
         _const_spec((1, 1)), _const_spec((32, 1)), _const_spec((1, 1))],
        _row_spec(be, 64), jax.ShapeDtypeStruct((e, 64), f32),
        interpret=interp,
    )(psrc, pdst, we['ln_g'].reshape(1, 8), we['ln_b'].reshape(1, 8),
      weg['wh'], weg['ws'], weg['bs'].reshape(1, 32), weg['wv'],
      weg['wsv'], weg['bsv'].reshape(1, 1))

    tab = tab0
    ones = jnp.ones((e,), f32)
    cnt = jnp.maximum(jax.ops.segment_sum(ones, dst, num_segments=n), 1.0)
    cnt16a = jnp.broadcast_to(cnt[:, None], (n, 16)) * 0.5
    cnt16b = cnt16a

    bl = 1000
    for lp in params['layers']:
        gs = tab[src]
        gd = tab[dst]
        g1, g2, g3 = lp['g1'], lp['g2'], lp['g3']
        wh = g1['wh']
        ws = g1['ws']
        msg = _pc(
            _edge_layer_body, e // bl,
            [_row_spec(bl, 128), _row_spec(bl, 128), _row_spec(bl, 64),
             _const_spec((32, 65)), _const_spec((1, 65)), _const_spec((32, 65)),
             _const_spec((32, 32)), _const_spec((32, 32)), _const_spec((32, 32)),
             _const_spec((65, 32)), _const_spec((1, 32)),
             _const_spec((65, 32)), _const_spec((32, 32)), _const_spec((1, 32)),
             _const_spec((32, 32)), _const_spec((32, 32)), _const_spec((32, 32)),
             _const_spec((1, 32)), _const_spec((32, 32)), _const_spec((32, 32)),
             _const_spec((1, 32)),
             _const_spec((32, 32)), _const_spec((32, 32)), _const_spec((32, 32)),
             _const_spec((1, 32)), _const_spec((32, 32)), _const_spec((32, 32)),
             _const_spec((1, 32))],
            _row_spec(bl, 128), jax.ShapeDtypeStruct((e, 128), f32),
            interpret=interp,
        )(gs, gd, esev,
          wh[0:32, :], wh[32:33, :], wh[33:65, :],
          ws[0:32, :], ws[32:64, :], ws[64:96, :], ws[96:161, :],
          g1['bs'].reshape(1, 32), g1['wv'], g1['wsv'],
          g1['bsv'].reshape(1, 32),
          g2['wh'], g2['ws'][0:32, :], g2['ws'][32:64, :],
          g2['bs'].reshape(1, 32), g2['wv'], g2['wsv'],
          g2['bsv'].reshape(1, 32),
          g3['wh'], g3['ws'][0:32, :], g3['ws'][32:64, :],
          g3['bs'].reshape(1, 32), g3['wv'], g3['wsv'],
          g3['bsv'].reshape(1, 32))

        aggfull = jax.ops.segment_sum(msg, dst, num_segments=n)
        p0 = aggfull * 0.5
        p1 = p0

        f1, f2 = lp['ff1'], lp['ff2']
        tab = _pc(
            _node_layer_body, n // bn,
            [_row_spec(bn, 128), _row_spec(bn, 128), _row_spec(bn, 128),
             _row_spec(bn, 16), _row_spec(bn, 16),
             _const_spec((1, 32)), _const_spec((1, 32)),
             _const_spec((32, 64)), _const_spec((32, 128)),
             _const_spec((64, 128)), _const_spec((1, 128)),
             _const_spec((64, 64)), _const_spec((128, 64)),
             _const_spec((1, 64)),
             _const_spec((64, 64)), _const_spec((128, 32)),
             _const_spec((64, 32)), _const_spec((1, 32)),
             _const_spec((64, 32)), _const_spec((32, 32)),
             _const_spec((1, 32)),
             _const_spec((1, 32)), _const_spec((1, 32))],
            _row_spec(bn, 128), jax.ShapeDtypeStruct((n, 128), f32),
            interpret=interp,
        )(tab, p0, p1, cnt16a, cnt16b,
          lp['ln1_g'].reshape(1, 32), lp['ln1_b'].reshape(1, 32),
          f1['wh'], f1['ws'][0:32, :], f1['ws'][32:96, :],
          f1['bs'].reshape(1, 128), f1['wv'], f1['wsv'],
          f1['bsv'].reshape(1, 64),
          f2['wh'], f2['ws'][0:128, :], f2['ws'][128:192, :],
          f2['bs'].reshape(1, 32), f2['wv'], f2['wsv'],
          f2['bsv'].reshape(1, 32),
          lp['ln2_g'].reshape(1, 32), lp['ln2_b'].reshape(1, 32))

    wo = params['Wout']
    wog = wo['gvp']
    pr = params['pred']
    batch2 = batch.reshape(n, 1)
    _, res = _pc(
        _head_body, n // bn,
        [_row_spec(bn, 128), _row_spec(bn, 1),
         _const_spec((1, 32)), _const_spec((1, 32)),
         _const_spec((32, 32)), _const_spec((32, 32)), _const_spec((32, 32)),
         _const_spec((1, 32)),
         _const_spec((32, 32)), _const_spec((1, 32)), _const_spec((32, 1)),
         _const_spec((1, 1))],
        [_const_spec((_NG, 32)), _const_spec((_NG, 1))],
        [jax.ShapeDtypeStruct((_NG, 32), f32),
         jax.ShapeDtypeStruct((_NG, 1), f32)],
        interpret=interp,
    )(tab, batch2,
      wo['ln_g'].reshape(1, 32), wo['ln_b'].reshape(1, 32),
      wog['wh'], wog['ws'][0:32, :], wog['ws'][32:64, :],
      wog['bs'].reshape(1, 32),
      pr['w1'], pr['b1'].reshape(1, 32), pr['w2'], pr['b2'].reshape(1, 1))
    return res
